# Initial kernel scaffold; baseline (speedup 1.0000x reference)
import jax
import jax.numpy as jnp
from jax.experimental import pallas as pl


def kernel(stimulus_frame, initial_spikes, spat_filters, timecourse_filters, feedback_filters, coupling_filters, bias, coupled_sel, stim_time):
    raise NotImplementedError("write your pallas kernel here")


# speedup vs baseline: 5.3733x; 5.3733x over previous
import functools

import jax
import jax.numpy as jnp
from jax.experimental import pallas as pl
from jax.experimental.pallas import tpu as pltpu

_HI = jax.lax.Precision.HIGHEST


def _stim_kernel(stim_flat_ref, spatT_ref, windows_ref, tcT_ref, biasT_ref,
                 out_ref):
    sf = jnp.dot(stim_flat_ref[...], spatT_ref[...],
                 preferred_element_type=jnp.float32, precision=_HI)
    convT = jnp.dot(windows_ref[...], tcT_ref[...],
                    preferred_element_type=jnp.float32, precision=_HI)
    out_ref[...] = (sf[:, None, :] * convT[None, :, :]
                    + biasT_ref[...][None, :, :])


def _row_select(block, r):
    rows = jax.lax.broadcasted_iota(jnp.int32, (1, 8, 1), 1)
    return jnp.sum(jnp.where(rows == r, block, 0.0), axis=1)


def _row_blend(block, r, row):
    rows = jax.lax.broadcasted_iota(jnp.int32, (1, 8, 1), 1)
    return jnp.where(rows == r, row[:, None, :], block)


def _sim_kernel(carry0T_ref, stimg_ref, onehot_ref, cfr_ref, fbr_ref,
                out_sp_ref, out_gs_ref, acc_ref, *, n_steps, nbf, maxc):
    B, _, N = carry0T_ref.shape
    acc_ref[...] = jnp.zeros(acc_ref.shape, jnp.float32)
    fbr = fbr_ref[...]
    pad = 112 - nbf

    def scatter(spikes, a):
        g = jax.lax.dot_general(spikes, onehot_ref[...],
                                (((1,), (0,)), ((), ())),
                                precision=_HI,
                                preferred_element_type=jnp.float32)
        g = g.reshape(B, maxc, N)
        contrib = spikes[:, None, :] * fbr[None, :, :]
        for c in range(maxc):
            contrib = contrib + g[:, c, None, :] * cfr_ref[c][None, :, :]
        buf = jnp.concatenate(
            [contrib, jnp.zeros((B, pad, N), jnp.float32)], axis=1)
        start = a + 1
        align = jax.lax.rem(start, 8)
        base = pl.multiple_of(start - align, 8)
        buf = pltpu.roll(buf, align, 1)
        cur = acc_ref[:, pl.ds(base, 112), :]
        acc_ref[:, pl.ds(base, 112), :] = cur + buf

    def warm_body(s, carry):
        r = jax.lax.rem(s, 8)
        base = pl.multiple_of(s - r, 8)
        col = _row_select(carry0T_ref[:, pl.ds(base, 8), :], r)
        scatter(col, s)
        return carry

    jax.lax.fori_loop(0, nbf, warm_body, 0)

    def main_body(t, carry):
        a = nbf + t
        ra = jax.lax.rem(a, 8)
        base_a = pl.multiple_of(a - ra, 8)
        rt = jax.lax.rem(t, 8)
        base_t = pl.multiple_of(t - rt, 8)
        gensig = (_row_select(acc_ref[:, pl.ds(base_a, 8), :], ra)
                  + _row_select(stimg_ref[:, pl.ds(base_t, 8), :], rt))
        spikes = jax.nn.sigmoid(gensig)
        gs_blk = out_gs_ref[:, pl.ds(base_t, 8), :]
        out_gs_ref[:, pl.ds(base_t, 8), :] = _row_blend(gs_blk, rt, gensig)
        sp_blk = out_sp_ref[:, pl.ds(base_t, 8), :]
        out_sp_ref[:, pl.ds(base_t, 8), :] = _row_blend(sp_blk, rt, spikes)
        scatter(spikes, a)
        return carry

    jax.lax.fori_loop(0, n_steps, main_body, 0)


def kernel(stimulus_frame, initial_spikes, spat_filters, timecourse_filters,
           feedback_filters, coupling_filters, bias, coupled_sel, stim_time):
    B = stimulus_frame.shape[0]
    N, nbf = timecourse_filters.shape
    n_bins = stim_time.shape[0]
    n_init = initial_spikes.shape[2]
    maxc = coupling_filters.shape[1]
    out_len = n_bins - nbf + 1
    n_steps = n_bins - n_init
    steps_pad = -(-n_steps // 8) * 8
    f32 = jnp.float32

    stim_flat = stimulus_frame.reshape(B, -1)
    spatT = spat_filters.T
    win_idx = jnp.arange(out_len)[:, None] + jnp.arange(nbf)[None, :]
    windows = stim_time[win_idx]
    windows = jnp.pad(windows, ((0, steps_pad - out_len), (0, 0)))
    tcT = timecourse_filters.T
    biasT = bias.T
    stimg = pl.pallas_call(
        _stim_kernel,
        out_shape=jax.ShapeDtypeStruct((B, steps_pad, N), f32),
    )(stim_flat, spatT, windows, tcT, biasT)

    carry0T = jnp.transpose(initial_spikes[:, :, n_init - nbf:], (0, 2, 1))
    nbf_pad = -(-nbf // 8) * 8
    carry0T = jnp.pad(carry0T, ((0, 0), (0, nbf_pad - nbf), (0, 0)))
    fbr = feedback_filters[:, ::-1].T
    cfr = jnp.transpose(coupling_filters[:, :, ::-1], (1, 2, 0))
    onehot = (coupled_sel.T.reshape(1, -1) == jnp.arange(N)[:, None]
              ).astype(f32)

    acc_rows = -(-(nbf + n_steps + 112) // 8) * 8
    out_sp, out_gs = pl.pallas_call(
        functools.partial(_sim_kernel, n_steps=n_steps, nbf=nbf, maxc=maxc),
        out_shape=(jax.ShapeDtypeStruct((B, steps_pad, N), f32),
                   jax.ShapeDtypeStruct((B, steps_pad, N), f32)),
        scratch_shapes=[pltpu.VMEM((B, acc_rows, N), f32)],
    )(carry0T, stimg, onehot, cfr, fbr)

    output_spikes = jnp.concatenate(
        [initial_spikes,
         jnp.transpose(out_sp[:, :n_steps, :], (0, 2, 1))], axis=2)
    generator_signal = jnp.transpose(out_gs[:, :n_steps, :], (0, 2, 1))
    return output_spikes, generator_signal


# speedup vs baseline: 10.5501x; 1.9634x over previous
import functools

import jax
import jax.numpy as jnp
from jax.experimental import pallas as pl
from jax.experimental.pallas import tpu as pltpu

_HI = jax.lax.Precision.HIGHEST


def _stim_kernel(stim_flat_ref, spatT_ref, windows_ref, tcT_ref, biasT_ref,
                 out_ref):
    sf = jnp.dot(stim_flat_ref[...], spatT_ref[...],
                 preferred_element_type=jnp.float32, precision=_HI)
    convT = jnp.dot(windows_ref[...], tcT_ref[...],
                    preferred_element_type=jnp.float32, precision=_HI)
    out_ref[...] = (sf[:, None, :] * convT[None, :, :]
                    + biasT_ref[...][None, :, :])


def _row_select(block, r):
    rows = jax.lax.broadcasted_iota(jnp.int32, (1, 8, 1), 1)
    return jnp.sum(jnp.where(rows == r, block, 0.0), axis=1)


def _row_blend(block, r, row):
    rows = jax.lax.broadcasted_iota(jnp.int32, (1, 8, 1), 1)
    return jnp.where(rows == r, row[:, None, :], block)


def _sim_kernel(carry0T_ref, stimg_ref, selq_ref, selhi_ref, cfr_ref, fbr_ref,
                out_sp_ref, out_gs_ref, acc_ref, *, n_steps, nbf, maxc):
    B, _, N = carry0T_ref.shape
    acc_ref[...] = jnp.zeros(acc_ref.shape, jnp.float32)
    fbr = fbr_ref[...]
    nq = N // 128
    pad = 112 - nbf

    def scatter(spikes, a):
        g = jnp.zeros((B, maxc * N), jnp.float32)
        for q in range(nq):
            idxq = jnp.broadcast_to(selq_ref[q], (B, maxc * N))
            gq = jnp.take_along_axis(spikes[:, q * 128:(q + 1) * 128], idxq,
                                     axis=1, mode="promise_in_bounds")
            hit = jnp.broadcast_to(selhi_ref[...], (B, maxc * N)) == q
            g = g + jnp.where(hit, gq, 0.0)
        g = g.reshape(B, maxc, N)
        contrib = spikes[:, None, :] * fbr[None, :, :]
        for c in range(maxc):
            contrib = contrib + g[:, c, None, :] * cfr_ref[c][None, :, :]
        buf = jnp.concatenate(
            [contrib, jnp.zeros((B, pad, N), jnp.float32)], axis=1)
        start = a + 1
        align = jax.lax.rem(start, 8)
        base = pl.multiple_of(start - align, 8)
        buf = pltpu.roll(buf, align, 1)
        cur = acc_ref[:, pl.ds(base, 112), :]
        acc_ref[:, pl.ds(base, 112), :] = cur + buf

    def warm_body(s, carry):
        r = jax.lax.rem(s, 8)
        base = pl.multiple_of(s - r, 8)
        col = _row_select(carry0T_ref[:, pl.ds(base, 8), :], r)
        scatter(col, s)
        return carry

    jax.lax.fori_loop(0, nbf, warm_body, 0)

    def main_body(t, carry):
        a = nbf + t
        ra = jax.lax.rem(a, 8)
        base_a = pl.multiple_of(a - ra, 8)
        rt = jax.lax.rem(t, 8)
        base_t = pl.multiple_of(t - rt, 8)
        gensig = (_row_select(acc_ref[:, pl.ds(base_a, 8), :], ra)
                  + _row_select(stimg_ref[:, pl.ds(base_t, 8), :], rt))
        spikes = jax.nn.sigmoid(gensig)
        gs_blk = out_gs_ref[:, pl.ds(base_t, 8), :]
        out_gs_ref[:, pl.ds(base_t, 8), :] = _row_blend(gs_blk, rt, gensig)
        sp_blk = out_sp_ref[:, pl.ds(base_t, 8), :]
        out_sp_ref[:, pl.ds(base_t, 8), :] = _row_blend(sp_blk, rt, spikes)
        scatter(spikes, a)
        return carry

    jax.lax.fori_loop(0, n_steps, main_body, 0)


def kernel(stimulus_frame, initial_spikes, spat_filters, timecourse_filters,
           feedback_filters, coupling_filters, bias, coupled_sel, stim_time):
    B = stimulus_frame.shape[0]
    N, nbf = timecourse_filters.shape
    n_bins = stim_time.shape[0]
    n_init = initial_spikes.shape[2]
    maxc = coupling_filters.shape[1]
    out_len = n_bins - nbf + 1
    n_steps = n_bins - n_init
    steps_pad = -(-n_steps // 8) * 8
    f32 = jnp.float32

    stim_flat = stimulus_frame.reshape(B, -1)
    spatT = spat_filters.T
    win_idx = jnp.arange(out_len)[:, None] + jnp.arange(nbf)[None, :]
    windows = stim_time[win_idx]
    windows = jnp.pad(windows, ((0, steps_pad - out_len), (0, 0)))
    tcT = timecourse_filters.T
    biasT = bias.T
    stimg = pl.pallas_call(
        _stim_kernel,
        out_shape=jax.ShapeDtypeStruct((B, steps_pad, N), f32),
    )(stim_flat, spatT, windows, tcT, biasT)

    carry0T = jnp.transpose(initial_spikes[:, :, n_init - nbf:], (0, 2, 1))
    nbf_pad = -(-nbf // 8) * 8
    carry0T = jnp.pad(carry0T, ((0, 0), (0, nbf_pad - nbf), (0, 0)))
    fbr = feedback_filters[:, ::-1].T
    cfr = jnp.transpose(coupling_filters[:, :, ::-1], (1, 2, 0))
    sel_flat = coupled_sel.T.reshape(-1).astype(jnp.int32)
    nq = N // 128
    selq = jnp.stack([jnp.clip(sel_flat - 128 * q, 0, 127)
                      for q in range(nq)])[:, None, :]
    selhi = (sel_flat // 128).reshape(1, -1)

    acc_rows = -(-(nbf + n_steps + 112) // 8) * 8
    out_sp, out_gs = pl.pallas_call(
        functools.partial(_sim_kernel, n_steps=n_steps, nbf=nbf, maxc=maxc),
        out_shape=(jax.ShapeDtypeStruct((B, steps_pad, N), f32),
                   jax.ShapeDtypeStruct((B, steps_pad, N), f32)),
        scratch_shapes=[pltpu.VMEM((B, acc_rows, N), f32)],
    )(carry0T, stimg, selq, selhi, cfr, fbr)

    output_spikes = jnp.concatenate(
        [initial_spikes,
         jnp.transpose(out_sp[:, :n_steps, :], (0, 2, 1))], axis=2)
    generator_signal = jnp.transpose(out_gs[:, :n_steps, :], (0, 2, 1))
    return output_spikes, generator_signal


# speedup vs baseline: 13.9572x; 1.3229x over previous
import functools

import jax
import jax.numpy as jnp
from jax.experimental import pallas as pl
from jax.experimental.pallas import tpu as pltpu

_HI = jax.lax.Precision.HIGHEST


def _stim_kernel(stim_flat_ref, spatT_ref, windows_ref, tcT_ref, biasT_ref,
                 out_ref):
    sf = jnp.dot(stim_flat_ref[...], spatT_ref[...],
                 preferred_element_type=jnp.float32, precision=_HI)
    convT = jnp.dot(windows_ref[...], tcT_ref[...],
                    preferred_element_type=jnp.float32, precision=_HI)
    out_ref[...] = (sf[:, None, :] * convT[None, :, :]
                    + biasT_ref[...][None, :, :])


def _row_select(block, r):
    rows = jax.lax.broadcasted_iota(jnp.int32, (1, 8, 1), 1)
    return jnp.sum(jnp.where(rows == r, block, 0.0), axis=1)


def _row_blend(block, r, row):
    rows = jax.lax.broadcasted_iota(jnp.int32, (1, 8, 1), 1)
    return jnp.where(rows == r, row[:, None, :], block)


def _sim_kernel(carry0T_ref, stimg_ref, selq_ref, selhi_ref, cfs_ref,
                out_sp_ref, out_gs_ref, acc_ref, g_ref, *,
                n_steps, nbf, maxc):
    B, _, N = carry0T_ref.shape
    acc_ref[...] = jnp.zeros(acc_ref.shape, jnp.float32)
    nq = N // 128
    nl = N // 128
    ext = 112

    def scatter(spikes, a):
        for c in range(maxc):
            gc = jnp.zeros((B, N), jnp.float32)
            for q in range(nq):
                idx = jnp.broadcast_to(selq_ref[q, c][None, :], (B, N))
                part = jnp.take_along_axis(
                    spikes[:, q * 128:(q + 1) * 128], idx, axis=1,
                    mode="promise_in_bounds")
                hit = jnp.broadcast_to(selhi_ref[c][None, :], (B, N)) == q
                gc = gc + jnp.where(hit, part, 0.0)
            g_ref[c] = gc
        g_ref[maxc] = spikes
        start = a + 1
        align = jax.lax.rem(start, 8)
        base = pl.multiple_of(start - align, 8)
        for j in range(ext // 8):
            for l in range(nl):
                lanes = slice(l * 128, (l + 1) * 128)
                tmp = acc_ref[:, pl.ds(base + 8 * j, 8), lanes]
                for c in range(maxc + 1):
                    w = cfs_ref[align, c, 8 * j:8 * j + 8, lanes
                                ].astype(jnp.float32)
                    gbc = jnp.broadcast_to(g_ref[c][:, None, lanes],
                                           (B, 8, 128))
                    tmp = tmp + gbc * w[None, :, :]
                acc_ref[:, pl.ds(base + 8 * j, 8), lanes] = tmp

    def warm_body(s, carry):
        r = jax.lax.rem(s, 8)
        base = pl.multiple_of(s - r, 8)
        col = _row_select(carry0T_ref[:, pl.ds(base, 8), :], r)
        scatter(col, s)
        return carry

    jax.lax.fori_loop(0, nbf, warm_body, 0)

    def main_body(t, carry):
        a = nbf + t
        ra = jax.lax.rem(a, 8)
        base_a = pl.multiple_of(a - ra, 8)
        rt = jax.lax.rem(t, 8)
        base_t = pl.multiple_of(t - rt, 8)
        gensig = (_row_select(acc_ref[:, pl.ds(base_a, 8), :], ra)
                  + _row_select(stimg_ref[:, pl.ds(base_t, 8), :], rt))
        spikes = jax.nn.sigmoid(gensig)
        gs_blk = out_gs_ref[:, pl.ds(base_t, 8), :]
        out_gs_ref[:, pl.ds(base_t, 8), :] = _row_blend(gs_blk, rt, gensig)
        sp_blk = out_sp_ref[:, pl.ds(base_t, 8), :]
        out_sp_ref[:, pl.ds(base_t, 8), :] = _row_blend(sp_blk, rt, spikes)
        scatter(spikes, a)
        return carry

    jax.lax.fori_loop(0, n_steps, main_body, 0)


def kernel(stimulus_frame, initial_spikes, spat_filters, timecourse_filters,
           feedback_filters, coupling_filters, bias, coupled_sel, stim_time):
    B = stimulus_frame.shape[0]
    N, nbf = timecourse_filters.shape
    n_bins = stim_time.shape[0]
    n_init = initial_spikes.shape[2]
    maxc = coupling_filters.shape[1]
    out_len = n_bins - nbf + 1
    n_steps = n_bins - n_init
    steps_pad = -(-n_steps // 8) * 8
    f32 = jnp.float32

    stim_flat = stimulus_frame.reshape(B, -1)
    spatT = spat_filters.T
    win_idx = jnp.arange(out_len)[:, None] + jnp.arange(nbf)[None, :]
    windows = stim_time[win_idx]
    windows = jnp.pad(windows, ((0, steps_pad - out_len), (0, 0)))
    tcT = timecourse_filters.T
    biasT = bias.T
    stimg = pl.pallas_call(
        _stim_kernel,
        out_shape=jax.ShapeDtypeStruct((B, steps_pad, N), f32),
    )(stim_flat, spatT, windows, tcT, biasT)

    carry0T = jnp.transpose(initial_spikes[:, :, n_init - nbf:], (0, 2, 1))
    nbf_pad = -(-nbf // 8) * 8
    carry0T = jnp.pad(carry0T, ((0, 0), (0, nbf_pad - nbf), (0, 0)))
    cfr = jnp.transpose(coupling_filters[:, :, ::-1], (1, 2, 0))
    fbr = feedback_filters[:, ::-1].T
    call = jnp.concatenate([cfr, fbr[None]], axis=0)
    ext = 112
    cfs = jnp.stack([jnp.pad(call, ((0, 0), (s, ext - nbf - s), (0, 0)))
                     for s in range(8)])
    cfs = cfs.astype(jnp.bfloat16)
    selT = coupled_sel.T.astype(jnp.int32)
    nq = N // 128
    selq = jnp.stack([jnp.clip(selT - 128 * q, 0, 127) for q in range(nq)])
    selhi = selT // 128

    acc_rows = -(-(nbf + n_steps + ext) // 8) * 8
    out_sp, out_gs = pl.pallas_call(
        functools.partial(_sim_kernel, n_steps=n_steps, nbf=nbf, maxc=maxc),
        out_shape=(jax.ShapeDtypeStruct((B, steps_pad, N), f32),
                   jax.ShapeDtypeStruct((B, steps_pad, N), f32)),
        scratch_shapes=[pltpu.VMEM((B, acc_rows, N), f32),
                        pltpu.VMEM((maxc + 1, B, N), f32)],
    )(carry0T, stimg, selq, selhi, cfs)

    output_spikes = jnp.concatenate(
        [initial_spikes,
         jnp.transpose(out_sp[:, :n_steps, :], (0, 2, 1))], axis=2)
    generator_signal = jnp.transpose(out_gs[:, :n_steps, :], (0, 2, 1))
    return output_spikes, generator_signal
